# Initial kernel scaffold; baseline (speedup 1.0000x reference)
import jax
import jax.numpy as jnp
from jax.experimental import pallas as pl


def kernel(encode_node_features, edge_index_comment_profile_to_comment, edge_index_data_profile_to_data, edge_index_comment_to_data, Wg1, asrc1, adst1, bg1, Wg2, asrc2, adst2, bg2, Wg3, asrc3, adst3, bg3, Wm1, bm1, Wm2):
    raise NotImplementedError("write your pallas kernel here")


# speedup vs baseline: 23.6836x; 23.6836x over previous
import jax
import jax.numpy as jnp
from jax import lax
from jax.experimental import pallas as pl
from jax.experimental.pallas import tpu as pltpu
from jax.experimental.pallas import tpu_sc as plsc

N = 10000
D = 128
E = 320000
NP = 10240
EREAL = E + N
NTILES = 32
EC = 10368
ETOT = NTILES * EC
EB = 128
NBLK = EC // EB
NC, NS = 2, 16
ROWS_PER_SUB = NP // NS


def _prep_body(x_ref, w_ref, av_ref, h1_ref, h2_ref, h3_ref, a_ref, amax_ref):
    i = pl.program_id(0)
    xb = x_ref[...]
    arows = []
    for g, h_ref in enumerate((h1_ref, h2_ref, h3_ref)):
        hg = jnp.dot(xb, w_ref[g], preferred_element_type=jnp.float32)
        h_ref[...] = hg
        arows.append(jnp.dot(hg, av_ref[g], preferred_element_type=jnp.float32))
        arows.append(jnp.dot(hg, av_ref[3 + g], preferred_element_type=jnp.float32))
    z = jnp.zeros_like(arows[0])
    a_ref[...] = jnp.stack([arows[0], arows[2], arows[4],
                            arows[1], arows[3], arows[5], z, z])
    cur = jnp.stack([jnp.full((D,), jnp.max(r), jnp.float32) for r in
                     (arows[0], arows[2], arows[4],
                      arows[1], arows[3], arows[5], z, z)])
    prev = jnp.where(i == 0, jnp.full_like(cur, -3.0e38), amax_ref[...])
    amax_ref[...] = jnp.maximum(prev, cur)


def _prep(x_pad, w_all, av_all):
    bn = 512
    grid = (NP // bn,)
    return pl.pallas_call(
        _prep_body,
        grid=grid,
        in_specs=[
            pl.BlockSpec((bn, D), lambda i: (i, 0)),
            pl.BlockSpec((3, D, D), lambda i: (0, 0, 0)),
            pl.BlockSpec((8, D), lambda i: (0, 0)),
        ],
        out_specs=[
            pl.BlockSpec((bn, D), lambda i: (i, 0)),
            pl.BlockSpec((bn, D), lambda i: (i, 0)),
            pl.BlockSpec((bn, D), lambda i: (i, 0)),
            pl.BlockSpec((8, bn), lambda i: (0, i)),
            pl.BlockSpec((8, D), lambda i: (0, 0)),
        ],
        out_shape=[
            jax.ShapeDtypeStruct((NP, D), jnp.float32),
            jax.ShapeDtypeStruct((NP, D), jnp.float32),
            jax.ShapeDtypeStruct((NP, D), jnp.float32),
            jax.ShapeDtypeStruct((8, NP), jnp.float32),
            jax.ShapeDtypeStruct((8, D), jnp.float32),
        ],
    )(x_pad, w_all, av_all)


def _sc_body(h1, h2, h3, a_hbm, amax_hbm, s1, s2, s3, d1, d2, d3,
             out_hbm, den_hbm,
             as_v, ad_v, exb_v, rows_v, sidx_v, didx_v, mrow_v,
             den0_v, u_sh, den_sh, sem):
    c = lax.axis_index("c")
    s = lax.axis_index("s")
    wid = s * NC + c
    h_refs = (h1, h2, h3)
    s_refs = (s1, s2, s3)
    d_refs = (d1, d2, d3)

    for g in range(3):
        pltpu.sync_copy(a_hbm.at[g], as_v)
        pltpu.sync_copy(a_hbm.at[3 + g], ad_v)
        pltpu.sync_copy(amax_hbm.at[g], mrow_v.at[0])
        pltpu.sync_copy(amax_hbm.at[3 + g], mrow_v.at[1])

        def _zero_rows(i, _):
            for k in range(D // 16):
                rows_v[i, pl.ds(16 * k, 16)] = jnp.zeros((16,), jnp.float32)
            return 0
        lax.fori_loop(0, EB, _zero_rows, 0)

        def _zero_den(i, _):
            den0_v[pl.ds(i * 16, 16)] = jnp.zeros((16,), jnp.float32)
            return 0
        lax.fori_loop(0, ROWS_PER_SUB // 16, _zero_den, 0)

        for j in range(ROWS_PER_SUB // EB):
            pltpu.sync_copy(rows_v, u_sh.at[pl.ds(s * ROWS_PER_SUB + j * EB, EB)])
        pltpu.sync_copy(den0_v, den_sh.at[pl.ds(s * ROWS_PER_SUB, ROWS_PER_SUB)])
        plsc.subcore_barrier()

        msum = mrow_v[0, pl.ds(0, 16)][0] + mrow_v[1, pl.ds(0, 16)][0]
        mbound = jnp.where(msum > 0, msum, 0.2 * msum)

        def _blk(b, _):
            base = b * EB
            pltpu.sync_copy(s_refs[g].at[pl.ds(wid * EC + base, EB)], sidx_v)
            pltpu.sync_copy(d_refs[g].at[pl.ds(wid * EC + base, EB)], didx_v)
            for j in range(EB // 16):
                si = sidx_v[pl.ds(j * 16, 16)]
                di = didx_v[pl.ds(j * 16, 16)]
                e = plsc.load_gather(as_v, [si]) + plsc.load_gather(ad_v, [di])
                e = jnp.where(e > 0, e, 0.2 * e)
                ex = jnp.exp(e - mbound)
                gid = wid * EC + base + j * 16 + lax.iota(jnp.int32, 16)
                ex = jnp.where(gid < EREAL, ex, 0.0)
                exb_v[pl.ds(j * 16, 16)] = ex
            pltpu.async_copy(h_refs[g].at[sidx_v], rows_v, sem).wait()

            def _scale(j, _):
                exv = exb_v[pl.ds(j * 16, 16)]
                for l in range(16):
                    i = j * 16 + l
                    sc = exv[l]
                    for k in range(D // 16):
                        rows_v[i, pl.ds(16 * k, 16)] = rows_v[i, pl.ds(16 * k, 16)] * sc
                return 0
            lax.fori_loop(0, EB // 16, _scale, 0)
            pltpu.sync_copy(rows_v, u_sh.at[didx_v], add=True)
            pltpu.sync_copy(exb_v, den_sh.at[didx_v], add=True)
            return 0
        lax.fori_loop(0, NBLK, _blk, 0)
        plsc.subcore_barrier()

        row0 = s * ROWS_PER_SUB
        slot = (2 * g + c) * NP
        pltpu.sync_copy(u_sh.at[pl.ds(row0, ROWS_PER_SUB)],
                        out_hbm.at[pl.ds(slot + row0, ROWS_PER_SUB)])
        pltpu.sync_copy(den_sh.at[pl.ds(row0, ROWS_PER_SUB)],
                        den_hbm.at[pl.ds(slot + row0, ROWS_PER_SUB)])


def _sc_edge_pass(h1, h2, h3, a_all, amax, s1, s2, s3, d1, d2, d3):
    mesh = plsc.VectorSubcoreMesh(core_axis_name="c", subcore_axis_name="s",
                                  num_cores=NC, num_subcores=NS)
    fn = pl.kernel(
        _sc_body,
        out_type=(jax.ShapeDtypeStruct((6 * NP, D), jnp.float32),
                  jax.ShapeDtypeStruct((8 * NP,), jnp.float32)),
        mesh=mesh,
        scratch_types=[
            pltpu.VMEM((NP,), jnp.float32),
            pltpu.VMEM((NP,), jnp.float32),
            pltpu.VMEM((EB,), jnp.float32),
            pltpu.VMEM((EB, D), jnp.float32),
            pltpu.VMEM((EB,), jnp.int32),
            pltpu.VMEM((EB,), jnp.int32),
            pltpu.VMEM((2, D), jnp.float32),
            pltpu.VMEM((ROWS_PER_SUB,), jnp.float32),
            pltpu.VMEM_SHARED((NP, D), jnp.float32),
            pltpu.VMEM_SHARED((NP,), jnp.float32),
            pltpu.SemaphoreType.DMA,
        ],
        compiler_params=pltpu.CompilerParams(needs_layout_passes=False),
    )
    return fn(h1, h2, h3, a_all, amax, s1, s2, s3, d1, d2, d3)


def _post_body(u_ref, den_ref, wm1_ref, p_ref, out_ref):
    u = u_ref[...]
    den = den_ref[...]
    wm1 = wm1_ref[...]
    p = p_ref[...]
    hs, ws = [], []
    for g in range(3):
        ug = u[2 * g] + u[2 * g + 1]
        dg = den[2 * g] + den[2 * g + 1]
        hg = ug / (dg[:, None] + 1e-16) + p[g][None, :]
        t = jnp.tanh(jnp.dot(hg, wm1, preferred_element_type=jnp.float32)
                     + p[3][None, :])
        ws.append(jnp.sum(t * p[4][None, :], axis=1, keepdims=True))
        hs.append(hg)
    wm = jnp.maximum(jnp.maximum(ws[0], ws[1]), ws[2])
    es = [jnp.exp(w - wm) for w in ws]
    ssum = es[0] + es[1] + es[2]
    acc = es[0] * hs[0] + es[1] * hs[1] + es[2] * hs[2]
    out_ref[...] = acc / ssum


def _post(u6, den8, wm1, p_all):
    bm = 256
    grid = (NP // bm,)
    return pl.pallas_call(
        _post_body,
        grid=grid,
        in_specs=[
            pl.BlockSpec((6, bm, D), lambda i: (0, i, 0)),
            pl.BlockSpec((8, bm), lambda i: (0, i)),
            pl.BlockSpec((D, D), lambda i: (0, 0)),
            pl.BlockSpec((8, D), lambda i: (0, 0)),
        ],
        out_specs=pl.BlockSpec((bm, D), lambda i: (i, 0)),
        out_shape=jax.ShapeDtypeStruct((NP, D), jnp.float32),
    )(u6, den8, wm1, p_all)


def kernel(encode_node_features,
           edge_index_comment_profile_to_comment,
           edge_index_data_profile_to_data,
           edge_index_comment_to_data,
           Wg1, asrc1, adst1, bg1,
           Wg2, asrc2, adst2, bg2,
           Wg3, asrc3, adst3, bg3,
           Wm1, bm1, Wm2):
    x = encode_node_features
    x_pad = jnp.zeros((NP, D), jnp.float32).at[:N].set(x)
    w_all = jnp.stack([Wg1, Wg2, Wg3])
    z = jnp.zeros((D,), jnp.float32)
    av_all = jnp.stack([asrc1, asrc2, asrc3, adst1, adst2, adst3, z, z])

    loops = jnp.arange(N, dtype=jnp.int32)
    pad = jnp.zeros((ETOT - EREAL,), jnp.int32)

    def edges(ei):
        src = jnp.concatenate([ei[0], loops, pad])
        dst = jnp.concatenate([ei[1], loops, pad])
        return src, dst

    s1, d1 = edges(edge_index_comment_profile_to_comment)
    s2, d2 = edges(edge_index_data_profile_to_data)
    s3, d3 = edges(edge_index_comment_to_data)

    h1, h2, h3, a_all, amax = _prep(x_pad, w_all, av_all)
    u_flat, den_flat = _sc_edge_pass(h1, h2, h3, a_all, amax,
                                     s1, s2, s3, d1, d2, d3)
    u6 = u_flat.reshape(6, NP, D)
    den8 = den_flat.reshape(8, NP)

    p_all = jnp.stack([bg1, bg2, bg3, bm1, Wm2[:, 0], z, z, z])
    out = _post(u6, den8, Wm1, p_all)
    return out[:N]
